# Initial kernel scaffold; baseline (speedup 1.0000x reference)
import jax
import jax.numpy as jnp
from jax.experimental import pallas as pl


def kernel(x):
    raise NotImplementedError("write your pallas kernel here")


# speedup vs baseline: 19.7720x; 19.7720x over previous
import functools

import jax
import jax.numpy as jnp
from jax.experimental import pallas as pl
from jax.experimental.pallas import tpu as pltpu

_TIMESTEP = 5
_VTH = 1.0
_TAU = 0.5
_BETA = 0.2

_B = 80
_C, _H, _W = 192, 32, 32
_P = _C * _H * _W
_K = int(_BETA * _P)
_BS = _B // _TIMESTEP

_ROWS_PER_BLK = 8
_LANES = 128
_SUBS = _P // _LANES
_SUB_CHUNK = 192
_NCHUNK = _SUBS // _SUB_CHUNK


def _monotone_key(xf):
    b = jax.lax.bitcast_convert_type(xf, jnp.int32)
    return jnp.where(b < 0, b ^ jnp.int32(0x7FFFFFFF), b)


def _threshold_body(x_ref, o_ref):
    key = _monotone_key(x_ref[...])
    cnt0 = jnp.sum((key >= 0).astype(jnp.int32), axis=1, keepdims=True)
    t = jnp.where(cnt0 >= _K, jnp.int32(0), jnp.iinfo(jnp.int32).min)
    for bit in range(30, -1, -1):
        cand = t + jnp.int32(1 << bit)
        cnt = jnp.sum((key >= cand).astype(jnp.int32), axis=1, keepdims=True)
        t = jnp.where(cnt >= _K, cand, t)
    o_ref[...] = jnp.broadcast_to(t.reshape(1, _ROWS_PER_BLK, 1),
                                  (1, _ROWS_PER_BLK, _LANES))


def _lif_body(thr_ref, x_ref, o_ref):
    j = pl.program_id(0)
    u = jnp.zeros((_SUB_CHUNK, _LANES), jnp.float32)
    for t in range(_TIMESTEP):
        xt = x_ref[t, 0]
        kth = thr_ref[t * _BS + j]
        mask = (_monotone_key(xt) >= kth).astype(jnp.float32)
        spk_prev = (u > _VTH).astype(jnp.float32)
        u = _TAU * u * (1.0 - spk_prev) + xt
        s = (u > _VTH).astype(jnp.float32)
        o_ref[t, 0] = s * mask


def kernel(x):
    flat = x.reshape(_B, _P)
    nblk = _B // _ROWS_PER_BLK
    thr = pl.pallas_call(
        _threshold_body,
        grid=(nblk,),
        in_specs=[pl.BlockSpec((_ROWS_PER_BLK, _P), lambda i: (i, 0))],
        out_specs=pl.BlockSpec((1, _ROWS_PER_BLK, _LANES), lambda i: (i, 0, 0)),
        out_shape=jax.ShapeDtypeStruct((nblk, _ROWS_PER_BLK, _LANES), jnp.int32),
    )(flat)
    thr80 = thr[:, :, 0].reshape(_B)

    x4 = x.reshape(_TIMESTEP, _BS, _SUBS, _LANES)
    out = pl.pallas_call(
        _lif_body,
        grid_spec=pltpu.PrefetchScalarGridSpec(
            num_scalar_prefetch=1,
            grid=(_BS, _NCHUNK),
            in_specs=[pl.BlockSpec((_TIMESTEP, 1, _SUB_CHUNK, _LANES),
                                   lambda j, c, *_: (0, j, c, 0))],
            out_specs=pl.BlockSpec((_TIMESTEP, 1, _SUB_CHUNK, _LANES),
                                   lambda j, c, *_: (0, j, c, 0)),
        ),
        out_shape=jax.ShapeDtypeStruct((_TIMESTEP, _BS, _SUBS, _LANES),
                                       jnp.float32),
    )(thr80, x4)
    return out.reshape(_B, _C, _H, _W)


# speedup vs baseline: 58.0355x; 2.9352x over previous
import functools

import jax
import jax.numpy as jnp
from jax.experimental import pallas as pl
from jax.experimental.pallas import tpu as pltpu

_TIMESTEP = 5
_VTH = 1.0
_TAU = 0.5
_BETA = 0.2

_B = 80
_C, _H, _W = 192, 32, 32
_P = _C * _H * _W
_K = int(_BETA * _P)
_BS = _B // _TIMESTEP
_HW = _H * _W

_ROWS_PER_BLK = 8
_HW_CHUNK = 512
_NCHUNK = _HW // _HW_CHUNK


def _monotone_key(xf):
    b = jax.lax.bitcast_convert_type(xf, jnp.int32)
    return jnp.where(b < 0, b ^ jnp.int32(0x7FFFFFFF), b)


def _threshold_body(x_ref, o_ref):
    key = _monotone_key(x_ref[...])
    cnt0 = jnp.sum((key >= 0).astype(jnp.int32), axis=(1, 2), keepdims=True)
    t = jnp.where(cnt0 >= _K, jnp.int32(0), jnp.iinfo(jnp.int32).min)
    for bit in range(30, -1, -1):
        cand = t + jnp.int32(1 << bit)
        cnt = jnp.sum((key >= cand).astype(jnp.int32), axis=(1, 2),
                      keepdims=True)
        t = jnp.where(cnt >= _K, cand, t)
    o_ref[...] = jnp.broadcast_to(t.reshape(1, _ROWS_PER_BLK, 1),
                                  (1, _ROWS_PER_BLK, 128))


def _lif_body(thr_ref, x_ref, o_ref):
    j = pl.program_id(0)
    u = jnp.zeros((_HW_CHUNK, _C), jnp.float32)
    for t in range(_TIMESTEP):
        xt = x_ref[t, 0]
        kth = thr_ref[t * _BS + j]
        mask = (_monotone_key(xt) >= kth).astype(jnp.float32)
        spk_prev = (u > _VTH).astype(jnp.float32)
        u = _TAU * u * (1.0 - spk_prev) + xt
        s = (u > _VTH).astype(jnp.float32)
        o_ref[t, 0] = s * mask


def kernel(x):
    xp = x.transpose(0, 2, 3, 1).reshape(_B, _HW, _C)
    nblk = _B // _ROWS_PER_BLK
    thr = pl.pallas_call(
        _threshold_body,
        grid=(nblk,),
        in_specs=[pl.BlockSpec((_ROWS_PER_BLK, _HW, _C), lambda i: (i, 0, 0))],
        out_specs=pl.BlockSpec((1, _ROWS_PER_BLK, 128), lambda i: (i, 0, 0)),
        out_shape=jax.ShapeDtypeStruct((nblk, _ROWS_PER_BLK, 128), jnp.int32),
    )(xp)
    thr80 = thr[:, :, 0].reshape(_B)

    x4 = xp.reshape(_TIMESTEP, _BS, _HW, _C)
    out = pl.pallas_call(
        _lif_body,
        grid_spec=pltpu.PrefetchScalarGridSpec(
            num_scalar_prefetch=1,
            grid=(_BS, _NCHUNK),
            in_specs=[pl.BlockSpec((_TIMESTEP, 1, _HW_CHUNK, _C),
                                   lambda j, c, *_: (0, j, c, 0))],
            out_specs=pl.BlockSpec((_TIMESTEP, 1, _HW_CHUNK, _C),
                                   lambda j, c, *_: (0, j, c, 0)),
        ),
        out_shape=jax.ShapeDtypeStruct((_TIMESTEP, _BS, _HW, _C),
                                       jnp.float32),
    )(thr80, x4)
    return out.reshape(_B, _H, _W, _C).transpose(0, 3, 1, 2)


# speedup vs baseline: 69.5279x; 1.1980x over previous
import jax
import jax.numpy as jnp
from jax.experimental import pallas as pl

_TIMESTEP = 5
_VTH = 1.0
_TAU = 0.5
_BETA = 0.2

_B = 80
_C, _H, _W = 192, 32, 32
_P = _C * _H * _W
_K = int(_BETA * _P)
_BS = _B // _TIMESTEP
_HW = _H * _W


def _monotone_key(xf):
    b = jax.lax.bitcast_convert_type(xf, jnp.int32)
    return jnp.where(b < 0, b ^ jnp.int32(0x7FFFFFFF), b)


def _fused_body(x_ref, o_ref):
    xb = x_ref[...].reshape(_TIMESTEP, _HW, _C)
    key = _monotone_key(xb)
    cnt0 = jnp.sum((key >= 0).astype(jnp.int32), axis=(1, 2), keepdims=True)
    thr = jnp.where(cnt0 >= _K, jnp.int32(0), jnp.iinfo(jnp.int32).min)
    for bit in range(30, -1, -1):
        cand = thr + jnp.int32(1 << bit)
        cnt = jnp.sum((key >= cand).astype(jnp.int32), axis=(1, 2),
                      keepdims=True)
        thr = jnp.where(cnt >= _K, cand, thr)
    u = jnp.zeros((_HW, _C), jnp.float32)
    for t in range(_TIMESTEP):
        mask = (key[t] >= thr[t]).astype(jnp.float32)
        spk_prev = (u > _VTH).astype(jnp.float32)
        u = _TAU * u * (1.0 - spk_prev) + xb[t]
        s = (u > _VTH).astype(jnp.float32)
        o_ref[t, 0] = s * mask


def kernel(x):
    xp = x.transpose(0, 2, 3, 1).reshape(_TIMESTEP, _BS, _HW, _C)
    out = pl.pallas_call(
        _fused_body,
        grid=(_BS,),
        in_specs=[pl.BlockSpec((_TIMESTEP, 1, _HW, _C),
                               lambda j: (0, j, 0, 0))],
        out_specs=pl.BlockSpec((_TIMESTEP, 1, _HW, _C),
                               lambda j: (0, j, 0, 0)),
        out_shape=jax.ShapeDtypeStruct((_TIMESTEP, _BS, _HW, _C),
                                       jnp.float32),
    )(xp)
    return out.reshape(_B, _H, _W, _C).transpose(0, 3, 1, 2)
